# baseline (speedup 1.0000x reference)
import functools

import jax
import jax.numpy as jnp
from jax import lax
from jax.experimental import pallas as pl
from jax.experimental.pallas import tpu as pltpu
from jax.experimental.pallas import tpu_sc as plsc

_BINS = [-1.8, -1.5429, -1.2857, -1.0286, -0.7714, -0.5143, -0.2571, 0.0,
         0.2571, 0.5143, 0.7714, 1.0286, 1.2857, 1.5429, 1.8]
_NB = 16
_N = 4096
_NT = 16
_CH = _N // _NT
_STRIDE = 2048
_L = 16

_i32 = jnp.int32

_DNUMS = lax.GatherDimensionNumbers(
    offset_dims=(), collapsed_slice_dims=(0,), start_index_map=(0,))


def _permute(v, idx):
    return lax.gather(v, idx[:, None], _DNUMS, slice_sizes=(1,),
                      mode=lax.GatherScatterMode.PROMISE_IN_BOUNDS)


def _body(x_hbm, mem_out, ord_out, cnt_out,
          idx_a, idx_b, y_a, y_b, mva, mvb, rka, rkb,
          pos_a, pos_b, gv_a, gv_b, tcnt, allc, shared, sem):
    s = lax.axis_index("s")
    base = s * _CH
    lane = lax.iota(_i32, _L)
    seg = lane * _L
    zeros = jnp.zeros((_L,), _i32)
    ones = jnp.ones((_L,), _i32)

    def incl_prefix(v):
        p = v
        for sh in (1, 2, 4, 8):
            g = _permute(p, jnp.maximum(lane - sh, 0))
            p = p + g * jnp.where(lane >= sh, ones, zeros)
        return p

    def bcast(v, k):
        return _permute(v, jnp.full((_L,), k, _i32))

    for h, iref in enumerate((idx_a, idx_b)):
        for rr in range(8):
            r = h * 8 + rr
            iref[pl.ds(rr * _L, _L)] = (base + seg + r) * _STRIDE
    g1 = pltpu.async_copy(x_hbm.at[idx_a], y_a, sem)
    g2 = pltpu.async_copy(x_hbm.at[idx_b], y_b, sem)
    g1.wait()
    g2.wait()

    cnt = [zeros] * _NB
    for r in range(_L):
        h, rr = divmod(r, 8)
        yv = (y_a if h == 0 else y_b)[pl.ds(rr * _L, _L)]
        mv = zeros
        for b in _BINS:
            mv = mv + jnp.where(yv > b, ones, zeros)
        rk = zeros
        for b in range(_NB):
            hi = jnp.where(mv == b, ones, zeros)
            rk = rk + hi * cnt[b]
            cnt[b] = cnt[b] + hi
        mref, rref, gref = (mva, rka, gv_a) if h == 0 else (mvb, rkb, gv_b)
        mref[pl.ds(rr * _L, _L)] = mv
        rref[pl.ds(rr * _L, _L)] = rk
        gref[pl.ds(rr * _L, _L)] = base + seg + r

    m1 = pltpu.async_copy(mva, mem_out.at[gv_a], sem)
    m2 = pltpu.async_copy(mvb, mem_out.at[gv_b], sem)

    excl = []
    tmine = zeros
    for b in range(_NB):
        ip = incl_prefix(cnt[b])
        excl.append(ip - cnt[b])
        tmine = tmine + bcast(ip, _L - 1) * jnp.where(lane == b, ones, zeros)

    tcnt[...] = tmine
    pltpu.sync_copy(tcnt, shared.at[pl.ds(s * _NB, _NB)])
    plsc.subcore_barrier()
    pltpu.sync_copy(shared, allc)

    tot = zeros
    prev = zeros
    svec = zeros + s
    for t in range(_NT):
        ct = allc[pl.ds(t * _NB, _NB)]
        tot = tot + ct
        prev = prev + ct * jnp.where(jnp.full((_L,), t, _i32) < svec, ones, zeros)
    ts_vec = incl_prefix(tot) - tot + prev

    m1.wait()
    m2.wait()

    for r in range(_L):
        h, rr = divmod(r, 8)
        mref, rref, pref = (mva, rka, pos_a) if h == 0 else (mvb, rkb, pos_b)
        mv = mref[pl.ds(rr * _L, _L)]
        p = rref[pl.ds(rr * _L, _L)] + _permute(ts_vec, mv)
        for b in range(_NB):
            p = p + excl[b] * jnp.where(mv == b, ones, zeros)
        pref[pl.ds(rr * _L, _L)] = p
    s1 = pltpu.async_copy(gv_a, ord_out.at[pos_a], sem)
    s2 = pltpu.async_copy(gv_b, ord_out.at[pos_b], sem)
    s1.wait()
    s2.wait()

    @pl.when(s == 0)
    def _():
        tcnt[...] = tot
        pltpu.sync_copy(tcnt, cnt_out)


@jax.jit
def kernel(x):
    xf = x.reshape(-1)
    run = functools.partial(
        pl.kernel,
        out_type=(
            jax.ShapeDtypeStruct((_N,), _i32),
            jax.ShapeDtypeStruct((_N,), _i32),
            jax.ShapeDtypeStruct((_NB,), _i32),
        ),
        mesh=plsc.VectorSubcoreMesh(core_axis_name="c", subcore_axis_name="s"),
        scratch_types=[
            pltpu.VMEM((128,), _i32),
            pltpu.VMEM((128,), _i32),
            pltpu.VMEM((128,), jnp.float32),
            pltpu.VMEM((128,), jnp.float32),
            pltpu.VMEM((128,), _i32),
            pltpu.VMEM((128,), _i32),
            pltpu.VMEM((128,), _i32),
            pltpu.VMEM((128,), _i32),
            pltpu.VMEM((128,), _i32),
            pltpu.VMEM((128,), _i32),
            pltpu.VMEM((128,), _i32),
            pltpu.VMEM((128,), _i32),
            pltpu.VMEM((_NB,), _i32),
            pltpu.VMEM((_NT * _NB,), _i32),
            pltpu.VMEM_SHARED((_NT * _NB,), _i32),
            pltpu.SemaphoreType.DMA,
        ],
    )(_body)
    memberships, order, counts = run(xf)
    return memberships, order, counts


# speedup vs baseline: 1.8954x; 1.8954x over previous
import functools

import jax
import jax.numpy as jnp
from jax import lax
from jax.experimental import pallas as pl
from jax.experimental.pallas import tpu as pltpu
from jax.experimental.pallas import tpu_sc as plsc

_BINS = [-1.8, -1.5429, -1.2857, -1.0286, -0.7714, -0.5143, -0.2571, 0.0,
         0.2571, 0.5143, 0.7714, 1.0286, 1.2857, 1.5429, 1.8]
_NB = 16
_N = 4096
_NT = 16
_CH = _N // _NT
_STRIDE = 2048
_L = 16

_i32 = jnp.int32

_DNUMS = lax.GatherDimensionNumbers(
    offset_dims=(), collapsed_slice_dims=(0,), start_index_map=(0,))


def _permute(v, idx):
    return lax.gather(v, idx[:, None], _DNUMS, slice_sizes=(1,),
                      mode=lax.GatherScatterMode.PROMISE_IN_BOUNDS)


def _body(x_hbm, mem_out, ord_out, cnt_out,
          idx_a, idx_b, y_a, y_b, mva, mvb, rka, rkb,
          pos_a, pos_b, tcnt, allc, shared, sem):
    s = lax.axis_index("s")
    base = s * _CH
    lane = lax.iota(_i32, _L)
    seg = lane * _L
    zeros = jnp.zeros((_L,), _i32)
    ones = jnp.ones((_L,), _i32)

    def incl_prefix(v):
        p = v
        for sh in (1, 2, 4, 8):
            g = _permute(p, jnp.maximum(lane - sh, 0))
            p = p + g * jnp.where(lane >= sh, ones, zeros)
        return p

    def bcast(v, k):
        return _permute(v, jnp.full((_L,), k, _i32))

    for h, iref in enumerate((idx_a, idx_b)):
        for rr in range(8):
            r = h * 8 + rr
            iref[pl.ds(rr * _L, _L)] = base + seg + r
    g1 = pltpu.async_copy(x_hbm.at[idx_a], y_a, sem)
    g2 = pltpu.async_copy(x_hbm.at[idx_b], y_b, sem)
    g1.wait()
    g2.wait()

    cnt = [zeros] * _NB
    for r in range(_L):
        h, rr = divmod(r, 8)
        yv = (y_a if h == 0 else y_b)[pl.ds(rr * _L, _L)]
        mv = zeros
        for b in _BINS:
            mv = mv + jnp.where(yv > b, ones, zeros)
        rk = zeros
        for b in range(_NB):
            hi = jnp.where(mv == b, ones, zeros)
            rk = rk + hi * cnt[b]
            cnt[b] = cnt[b] + hi
        mref, rref = (mva, rka) if h == 0 else (mvb, rkb)
        mref[pl.ds(rr * _L, _L)] = mv
        rref[pl.ds(rr * _L, _L)] = rk

    m1 = pltpu.async_copy(mva, mem_out.at[idx_a], sem)
    m2 = pltpu.async_copy(mvb, mem_out.at[idx_b], sem)

    excl = []
    tmine = zeros
    for b in range(_NB):
        ip = incl_prefix(cnt[b])
        excl.append(ip - cnt[b])
        tmine = tmine + bcast(ip, _L - 1) * jnp.where(lane == b, ones, zeros)

    tcnt[...] = tmine
    pltpu.sync_copy(tcnt, shared.at[pl.ds(s * _NB, _NB)])
    plsc.subcore_barrier()
    pltpu.sync_copy(shared, allc)

    tot = zeros
    prev = zeros
    svec = zeros + s
    for t in range(_NT):
        ct = allc[pl.ds(t * _NB, _NB)]
        tot = tot + ct
        prev = prev + ct * jnp.where(jnp.full((_L,), t, _i32) < svec, ones, zeros)
    ts_vec = incl_prefix(tot) - tot + prev

    m1.wait()
    m2.wait()

    for r in range(_L):
        h, rr = divmod(r, 8)
        mref, rref, pref = (mva, rka, pos_a) if h == 0 else (mvb, rkb, pos_b)
        mv = mref[pl.ds(rr * _L, _L)]
        p = rref[pl.ds(rr * _L, _L)] + _permute(ts_vec, mv)
        for b in range(_NB):
            p = p + excl[b] * jnp.where(mv == b, ones, zeros)
        pref[pl.ds(rr * _L, _L)] = p
    s1 = pltpu.async_copy(idx_a, ord_out.at[pos_a], sem)
    s2 = pltpu.async_copy(idx_b, ord_out.at[pos_b], sem)
    s1.wait()
    s2.wait()

    @pl.when(s == 0)
    def _():
        tcnt[...] = tot
        pltpu.sync_copy(tcnt, cnt_out)


@jax.jit
def kernel(x):
    xf = x[0, :, 0]
    run = functools.partial(
        pl.kernel,
        out_type=(
            jax.ShapeDtypeStruct((_N,), _i32),
            jax.ShapeDtypeStruct((_N,), _i32),
            jax.ShapeDtypeStruct((_NB,), _i32),
        ),
        mesh=plsc.VectorSubcoreMesh(core_axis_name="c", subcore_axis_name="s"),
        scratch_types=[
            pltpu.VMEM((128,), _i32),
            pltpu.VMEM((128,), _i32),
            pltpu.VMEM((128,), jnp.float32),
            pltpu.VMEM((128,), jnp.float32),
            pltpu.VMEM((128,), _i32),
            pltpu.VMEM((128,), _i32),
            pltpu.VMEM((128,), _i32),
            pltpu.VMEM((128,), _i32),
            pltpu.VMEM((128,), _i32),
            pltpu.VMEM((128,), _i32),
            pltpu.VMEM((_NB,), _i32),
            pltpu.VMEM((_NT * _NB,), _i32),
            pltpu.VMEM_SHARED((_NT * _NB,), _i32),
            pltpu.SemaphoreType.DMA,
        ],
    )(_body)
    memberships, order, counts = run(xf)
    return memberships, order, counts


# speedup vs baseline: 8.1222x; 4.2851x over previous
import functools

import jax
import jax.numpy as jnp
from jax import lax
from jax.experimental import pallas as pl
from jax.experimental.pallas import tpu as pltpu
from jax.experimental.pallas import tpu_sc as plsc

_BINS = [-1.8, -1.5429, -1.2857, -1.0286, -0.7714, -0.5143, -0.2571, 0.0,
         0.2571, 0.5143, 0.7714, 1.0286, 1.2857, 1.5429, 1.8]
_NB = 16
_N = 4096
_NT = 16
_CH = _N // _NT
_L = 16
_NV = _CH // _L

_i32 = jnp.int32

_DNUMS = lax.GatherDimensionNumbers(
    offset_dims=(), collapsed_slice_dims=(0,), start_index_map=(0,))


def _permute(v, idx):
    return lax.gather(v, idx[:, None], _DNUMS, slice_sizes=(1,),
                      mode=lax.GatherScatterMode.PROMISE_IN_BOUNDS)


def _body(y_hbm, mem_out, ord_out, cnt_out,
          y_ref, mnat, rks, gv_a, gv_b, pos_a, pos_b,
          tcnt, allc, shared_cnt, shared_ord, sem):
    s = lax.axis_index("s")
    base = s * _CH
    lane = lax.iota(_i32, _L)
    zeros = jnp.zeros((_L,), _i32)
    ones = jnp.ones((_L,), _i32)
    gt = [jnp.where(lane > k, ones, zeros) for k in range(_L - 1)] + [zeros]

    def incl_prefix(v):
        p = v
        for sh in (1, 2, 4, 8):
            g = _permute(p, jnp.maximum(lane - sh, 0))
            p = p + g * jnp.where(lane >= sh, ones, zeros)
        return p

    pltpu.sync_copy(y_hbm.at[pl.ds(base, _CH)], y_ref)

    rcnt = zeros
    for j in range(_NV):
        yv = y_ref[pl.ds(j * _L, _L)]
        mv = zeros
        for b in _BINS:
            mv = mv + jnp.where(yv > b, ones, zeros)
        crk = _permute(rcnt, mv)
        hist = zeros
        for k in range(_L):
            bl = _permute(mv, jnp.full((_L,), k, _i32))
            crk = crk + jnp.where(mv == bl, ones, zeros) * gt[k]
            hist = hist + jnp.where(lane == bl, ones, zeros)
        rcnt = rcnt + hist
        mnat[pl.ds(j * _L, _L)] = mv
        rks[pl.ds(j * _L, _L)] = crk
        half, jj = divmod(j, _NV // 2)
        gref = gv_a if half == 0 else gv_b
        gref[pl.ds(jj * _L, _L)] = base + j * _L + lane

    m1 = pltpu.async_copy(mnat, mem_out.at[pl.ds(base, _CH)], sem)

    tcnt[...] = rcnt
    pltpu.sync_copy(tcnt, shared_cnt.at[pl.ds(s * _NB, _NB)])
    plsc.subcore_barrier()
    pltpu.sync_copy(shared_cnt, allc)

    tot = zeros
    prev = zeros
    svec = zeros + s
    for t in range(_NT):
        ct = allc[pl.ds(t * _NB, _NB)]
        tot = tot + ct
        prev = prev + ct * jnp.where(jnp.full((_L,), t, _i32) < svec, ones, zeros)
    ts_vec = incl_prefix(tot) - tot + prev

    for j in range(_NV):
        mv = mnat[pl.ds(j * _L, _L)]
        p = rks[pl.ds(j * _L, _L)] + _permute(ts_vec, mv)
        half, jj = divmod(j, _NV // 2)
        pref = pos_a if half == 0 else pos_b
        pref[pl.ds(jj * _L, _L)] = p
    s1 = pltpu.async_copy(gv_a, shared_ord.at[pos_a], sem)
    s2 = pltpu.async_copy(gv_b, shared_ord.at[pos_b], sem)
    m1.wait()
    s1.wait()
    s2.wait()
    plsc.subcore_barrier()

    pltpu.sync_copy(shared_ord.at[pl.ds(base, _CH)], ord_out.at[pl.ds(base, _CH)])

    @pl.when(s == 0)
    def _():
        tcnt[...] = tot
        pltpu.sync_copy(tcnt, cnt_out)


@jax.jit
def kernel(x):
    xf = x[0, :, 0]
    run = functools.partial(
        pl.kernel,
        out_type=(
            jax.ShapeDtypeStruct((_N,), _i32),
            jax.ShapeDtypeStruct((_N,), _i32),
            jax.ShapeDtypeStruct((_NB,), _i32),
        ),
        mesh=plsc.VectorSubcoreMesh(core_axis_name="c", subcore_axis_name="s"),
        scratch_types=[
            pltpu.VMEM((_CH,), jnp.float32),
            pltpu.VMEM((_CH,), _i32),
            pltpu.VMEM((_CH,), _i32),
            pltpu.VMEM((128,), _i32),
            pltpu.VMEM((128,), _i32),
            pltpu.VMEM((128,), _i32),
            pltpu.VMEM((128,), _i32),
            pltpu.VMEM((_NB,), _i32),
            pltpu.VMEM((_NT * _NB,), _i32),
            pltpu.VMEM_SHARED((_NT * _NB,), _i32),
            pltpu.VMEM_SHARED((_N,), _i32),
            pltpu.SemaphoreType.DMA,
        ],
    )(_body)
    memberships, order, counts = run(xf)
    return memberships, order, counts


# speedup vs baseline: 8.4895x; 1.0452x over previous
import functools

import jax
import jax.numpy as jnp
from jax import lax
from jax.experimental import pallas as pl
from jax.experimental.pallas import tpu as pltpu
from jax.experimental.pallas import tpu_sc as plsc

_BINS = [-1.8, -1.5429, -1.2857, -1.0286, -0.7714, -0.5143, -0.2571, 0.0,
         0.2571, 0.5143, 0.7714, 1.0286, 1.2857, 1.5429, 1.8]
_NB = 16
_N = 4096
_NT = 16
_CH = _N // _NT
_L = 16
_NV = _CH // _L

_i32 = jnp.int32

_DNUMS = lax.GatherDimensionNumbers(
    offset_dims=(), collapsed_slice_dims=(0,), start_index_map=(0,))


def _permute(v, idx):
    return lax.gather(v, idx[:, None], _DNUMS, slice_sizes=(1,),
                      mode=lax.GatherScatterMode.PROMISE_IN_BOUNDS)


def _body(y_hbm, mem_out, ord_out, cnt_out,
          y_ref, mnat, rks, gv_a, gv_b, pos_a, pos_b,
          tcnt, allc, shared_cnt, shared_ord, sem):
    s = lax.axis_index("s")
    base = s * _CH
    lane = lax.iota(_i32, _L)
    zeros = jnp.zeros((_L,), _i32)
    ones = jnp.ones((_L,), _i32)
    fifteen = jnp.full((_L,), 15, _i32)
    shl_lane = (lane & 7) * 4
    mlow_lane = jnp.where(lane < 8, ones, zeros)

    def incl_prefix(v):
        p = v
        for sh in (1, 2, 4, 8):
            g = _permute(p, jnp.maximum(lane - sh, 0))
            p = p + g * jnp.where(lane >= sh, ones, zeros)
        return p

    pltpu.sync_copy(y_hbm.at[pl.ds(base, _CH)], y_ref)

    rcnt = zeros
    for j in range(_NV):
        yv = y_ref[pl.ds(j * _L, _L)]
        mv = zeros
        for b in _BINS:
            mv = mv + jnp.where(yv > b, ones, zeros)
        mlow = jnp.where(mv < 8, ones, zeros)
        sh = (mv & 7) * 4
        ohA = lax.shift_left(mlow, sh)
        ohB = lax.shift_left(ones - mlow, sh)
        exA = incl_prefix(ohA) - ohA
        exB = incl_prefix(ohB) - ohB
        crk_in = (mlow * (lax.shift_right_logical(exA, sh) & fifteen)
                  + (ones - mlow) * (lax.shift_right_logical(exB, sh) & fifteen))
        crk = crk_in + _permute(rcnt, mv)
        ta = _permute(exA, fifteen)
        tb = _permute(exB, fifteen)
        hist = (mlow_lane * (lax.shift_right_logical(ta, shl_lane) & fifteen)
                + (ones - mlow_lane) * (lax.shift_right_logical(tb, shl_lane) & fifteen)
                + jnp.where(lane == _permute(mv, fifteen), ones, zeros))
        rcnt = rcnt + hist
        mnat[pl.ds(j * _L, _L)] = mv
        rks[pl.ds(j * _L, _L)] = crk
        half, jj = divmod(j, _NV // 2)
        gref = gv_a if half == 0 else gv_b
        gref[pl.ds(jj * _L, _L)] = base + j * _L + lane

    m1 = pltpu.async_copy(mnat, mem_out.at[pl.ds(base, _CH)], sem)

    tcnt[...] = rcnt
    pltpu.sync_copy(tcnt, shared_cnt.at[pl.ds(s * _NB, _NB)])
    plsc.subcore_barrier()
    pltpu.sync_copy(shared_cnt, allc)

    tot = zeros
    prev = zeros
    svec = zeros + s
    for t in range(_NT):
        ct = allc[pl.ds(t * _NB, _NB)]
        tot = tot + ct
        prev = prev + ct * jnp.where(jnp.full((_L,), t, _i32) < svec, ones, zeros)
    ts_vec = incl_prefix(tot) - tot + prev

    for j in range(_NV):
        mv = mnat[pl.ds(j * _L, _L)]
        p = rks[pl.ds(j * _L, _L)] + _permute(ts_vec, mv)
        half, jj = divmod(j, _NV // 2)
        pref = pos_a if half == 0 else pos_b
        pref[pl.ds(jj * _L, _L)] = p
    s1 = pltpu.async_copy(gv_a, shared_ord.at[pos_a], sem)
    s2 = pltpu.async_copy(gv_b, shared_ord.at[pos_b], sem)
    m1.wait()
    s1.wait()
    s2.wait()
    plsc.subcore_barrier()

    pltpu.sync_copy(shared_ord.at[pl.ds(base, _CH)], ord_out.at[pl.ds(base, _CH)])

    @pl.when(s == 0)
    def _():
        tcnt[...] = tot
        pltpu.sync_copy(tcnt, cnt_out)


@jax.jit
def kernel(x):
    xf = x[0, :, 0]
    run = functools.partial(
        pl.kernel,
        out_type=(
            jax.ShapeDtypeStruct((_N,), _i32),
            jax.ShapeDtypeStruct((_N,), _i32),
            jax.ShapeDtypeStruct((_NB,), _i32),
        ),
        mesh=plsc.VectorSubcoreMesh(core_axis_name="c", subcore_axis_name="s"),
        scratch_types=[
            pltpu.VMEM((_CH,), jnp.float32),
            pltpu.VMEM((_CH,), _i32),
            pltpu.VMEM((_CH,), _i32),
            pltpu.VMEM((128,), _i32),
            pltpu.VMEM((128,), _i32),
            pltpu.VMEM((128,), _i32),
            pltpu.VMEM((128,), _i32),
            pltpu.VMEM((_NB,), _i32),
            pltpu.VMEM((_NT * _NB,), _i32),
            pltpu.VMEM_SHARED((_NT * _NB,), _i32),
            pltpu.VMEM_SHARED((_N,), _i32),
            pltpu.SemaphoreType.DMA,
        ],
    )(_body)
    memberships, order, counts = run(xf)
    return memberships, order, counts


# speedup vs baseline: 8.7409x; 1.0296x over previous
import functools

import jax
import jax.numpy as jnp
from jax import lax
from jax.experimental import pallas as pl
from jax.experimental.pallas import tpu as pltpu
from jax.experimental.pallas import tpu_sc as plsc

_BINS = [-1.8, -1.5429, -1.2857, -1.0286, -0.7714, -0.5143, -0.2571, 0.0,
         0.2571, 0.5143, 0.7714, 1.0286, 1.2857, 1.5429, 1.8]
_NB = 16
_N = 4096
_NT = 16
_CH = _N // _NT
_L = 16
_NV = _CH // _L

_i32 = jnp.int32

_DNUMS = lax.GatherDimensionNumbers(
    offset_dims=(), collapsed_slice_dims=(0,), start_index_map=(0,))


def _permute(v, idx):
    return lax.gather(v, idx[:, None], _DNUMS, slice_sizes=(1,),
                      mode=lax.GatherScatterMode.PROMISE_IN_BOUNDS)


def _body(y_hbm, mem_out, ord_out, cnt_out,
          y_ref, mnat, rks, gv_a, gv_b, pos_a, pos_b,
          tcnt, allc, shared_cnt, shared_ord, sem):
    s = lax.axis_index("s")
    base = s * _CH
    lane = lax.iota(_i32, _L)
    zeros = jnp.zeros((_L,), _i32)
    ones = jnp.ones((_L,), _i32)
    fifteen = jnp.full((_L,), 15, _i32)
    shl_lane = (lane & 7) * 4
    mlow_lane = jnp.where(lane < 8, ones, zeros)

    def incl_prefix(v):
        p = v
        for sh in (1, 2, 4, 8):
            g = _permute(p, jnp.maximum(lane - sh, 0))
            p = p + g * jnp.where(lane >= sh, ones, zeros)
        return p

    pltpu.sync_copy(y_hbm.at[pl.ds(base, _CH)], y_ref)

    def p1_body(j, rcnt):
        yv = y_ref[pl.ds(j * _L, _L)]
        mv = zeros
        for b in _BINS:
            mv = mv + jnp.where(yv > b, ones, zeros)
        mlow = jnp.where(mv < 8, ones, zeros)
        sh = (mv & 7) * 4
        ohA = lax.shift_left(mlow, sh)
        ohB = lax.shift_left(ones - mlow, sh)
        exA = incl_prefix(ohA) - ohA
        exB = incl_prefix(ohB) - ohB
        crk_in = (mlow * (lax.shift_right_logical(exA, sh) & fifteen)
                  + (ones - mlow) * (lax.shift_right_logical(exB, sh) & fifteen))
        crk = crk_in + _permute(rcnt, mv)
        ta = _permute(exA, fifteen)
        tb = _permute(exB, fifteen)
        hist = (mlow_lane * (lax.shift_right_logical(ta, shl_lane) & fifteen)
                + (ones - mlow_lane) * (lax.shift_right_logical(tb, shl_lane) & fifteen)
                + jnp.where(lane == _permute(mv, fifteen), ones, zeros))
        mnat[pl.ds(j * _L, _L)] = mv
        rks[pl.ds(j * _L, _L)] = crk
        return rcnt + hist

    rcnt = lax.fori_loop(0, _NV, p1_body, zeros)
    for j in range(_NV):
        half, jj = divmod(j, _NV // 2)
        gref = gv_a if half == 0 else gv_b
        gref[pl.ds(jj * _L, _L)] = base + j * _L + lane

    m1 = pltpu.async_copy(mnat, mem_out.at[pl.ds(base, _CH)], sem)

    tcnt[...] = rcnt
    pltpu.sync_copy(tcnt, shared_cnt.at[pl.ds(s * _NB, _NB)])
    plsc.subcore_barrier()
    pltpu.sync_copy(shared_cnt, allc)

    tot = zeros
    prev = zeros
    svec = zeros + s
    for t in range(_NT):
        ct = allc[pl.ds(t * _NB, _NB)]
        tot = tot + ct
        prev = prev + ct * jnp.where(jnp.full((_L,), t, _i32) < svec, ones, zeros)
    ts_vec = incl_prefix(tot) - tot + prev

    for half, pref in ((0, pos_a), (1, pos_b)):
        def p2_body(jj, _, pref=pref, off=half * (_NV // 2)):
            j = jj + off
            mv = mnat[pl.ds(j * _L, _L)]
            pref[pl.ds(jj * _L, _L)] = (rks[pl.ds(j * _L, _L)]
                                        + _permute(ts_vec, mv))
            return 0
        lax.fori_loop(0, _NV // 2, p2_body, 0)
    s1 = pltpu.async_copy(gv_a, shared_ord.at[pos_a], sem)
    s2 = pltpu.async_copy(gv_b, shared_ord.at[pos_b], sem)
    m1.wait()
    s1.wait()
    s2.wait()
    plsc.subcore_barrier()

    pltpu.sync_copy(shared_ord.at[pl.ds(base, _CH)], ord_out.at[pl.ds(base, _CH)])

    @pl.when(s == 0)
    def _():
        tcnt[...] = tot
        pltpu.sync_copy(tcnt, cnt_out)


@jax.jit
def kernel(x):
    xf = x[0, :, 0]
    run = functools.partial(
        pl.kernel,
        out_type=(
            jax.ShapeDtypeStruct((_N,), _i32),
            jax.ShapeDtypeStruct((_N,), _i32),
            jax.ShapeDtypeStruct((_NB,), _i32),
        ),
        mesh=plsc.VectorSubcoreMesh(core_axis_name="c", subcore_axis_name="s"),
        scratch_types=[
            pltpu.VMEM((_CH,), jnp.float32),
            pltpu.VMEM((_CH,), _i32),
            pltpu.VMEM((_CH,), _i32),
            pltpu.VMEM((128,), _i32),
            pltpu.VMEM((128,), _i32),
            pltpu.VMEM((128,), _i32),
            pltpu.VMEM((128,), _i32),
            pltpu.VMEM((_NB,), _i32),
            pltpu.VMEM((_NT * _NB,), _i32),
            pltpu.VMEM_SHARED((_NT * _NB,), _i32),
            pltpu.VMEM_SHARED((_N,), _i32),
            pltpu.SemaphoreType.DMA,
        ],
    )(_body)
    memberships, order, counts = run(xf)
    return memberships, order, counts


# speedup vs baseline: 8.7998x; 1.0067x over previous
import functools

import jax
import jax.numpy as jnp
from jax import lax
from jax.experimental import pallas as pl
from jax.experimental.pallas import tpu as pltpu
from jax.experimental.pallas import tpu_sc as plsc

_BINS = [-1.8, -1.5429, -1.2857, -1.0286, -0.7714, -0.5143, -0.2571, 0.0,
         0.2571, 0.5143, 0.7714, 1.0286, 1.2857, 1.5429, 1.8]
_NB = 16
_N = 4096
_NT = 16
_CH = _N // _NT
_L = 16
_NV = _CH // _L

_i32 = jnp.int32

_DNUMS = lax.GatherDimensionNumbers(
    offset_dims=(), collapsed_slice_dims=(0,), start_index_map=(0,))


def _permute(v, idx):
    return lax.gather(v, idx[:, None], _DNUMS, slice_sizes=(1,),
                      mode=lax.GatherScatterMode.PROMISE_IN_BOUNDS)


def _body(y_hbm, mem_out, ord_out, cnt_out,
          y_ref, mnat, rks, gv_a, gv_b, pos_a, pos_b,
          tcnt, allc, shared_cnt, shared_ord, sem):
    s = lax.axis_index("s")
    base = s * _CH
    lane = lax.iota(_i32, _L)
    zeros = jnp.zeros((_L,), _i32)
    ones = jnp.ones((_L,), _i32)
    fifteen = jnp.full((_L,), 15, _i32)
    shl_lane = (lane & 7) * 4
    mlow_lane = jnp.where(lane < 8, ones, zeros)

    def incl_prefix(v):
        p = v
        for sh in (1, 2, 4, 8):
            g = _permute(p, jnp.maximum(lane - sh, 0))
            p = p + g * jnp.where(lane >= sh, ones, zeros)
        return p

    pltpu.sync_copy(y_hbm.at[pl.ds(base, _CH)], y_ref)

    def p1_body(j, rcnt):
        yv = y_ref[pl.ds(j * _L, _L)]
        mv = zeros
        for b in _BINS:
            mv = mv + jnp.where(yv > b, ones, zeros)
        mlow = jnp.where(mv < 8, ones, zeros)
        sh = (mv & 7) * 4
        ohA = lax.shift_left(mlow, sh)
        ohB = lax.shift_left(ones - mlow, sh)
        exA = incl_prefix(ohA) - ohA
        exB = incl_prefix(ohB) - ohB
        crk_in = (mlow * (lax.shift_right_logical(exA, sh) & fifteen)
                  + (ones - mlow) * (lax.shift_right_logical(exB, sh) & fifteen))
        crk = crk_in + _permute(rcnt, mv)
        ta = _permute(exA, fifteen)
        tb = _permute(exB, fifteen)
        hist = (mlow_lane * (lax.shift_right_logical(ta, shl_lane) & fifteen)
                + (ones - mlow_lane) * (lax.shift_right_logical(tb, shl_lane) & fifteen)
                + jnp.where(lane == _permute(mv, fifteen), ones, zeros))
        mnat[pl.ds(j * _L, _L)] = mv
        rks[pl.ds(j * _L, _L)] = crk
        return rcnt + hist

    rcnt = lax.fori_loop(0, _NV, p1_body, zeros)
    for j in range(_NV):
        half, jj = divmod(j, _NV // 2)
        gref = gv_a if half == 0 else gv_b
        gref[pl.ds(jj * _L, _L)] = base + j * _L + lane

    m1 = pltpu.async_copy(mnat, mem_out.at[pl.ds(base, _CH)], sem)

    tcnt[...] = rcnt
    pltpu.sync_copy(tcnt, shared_cnt.at[pl.ds(s * _NB, _NB)])
    plsc.subcore_barrier()
    pltpu.sync_copy(shared_cnt, allc)

    tot = zeros
    prev = zeros
    svec = zeros + s
    for t in range(_NT):
        ct = allc[pl.ds(t * _NB, _NB)]
        tot = tot + ct
        prev = prev + ct * jnp.where(jnp.full((_L,), t, _i32) < svec, ones, zeros)
    ts_vec = incl_prefix(tot) - tot + prev

    @pl.when(s == 0)
    def _():
        tcnt[...] = tot
        pltpu.sync_copy(tcnt, cnt_out)

    for half, pref in ((0, pos_a), (1, pos_b)):
        def p2_body(jj, _, pref=pref, off=half * (_NV // 2)):
            j = jj + off
            mv = mnat[pl.ds(j * _L, _L)]
            pref[pl.ds(jj * _L, _L)] = (rks[pl.ds(j * _L, _L)]
                                        + _permute(ts_vec, mv))
            return 0
        lax.fori_loop(0, _NV // 2, p2_body, 0)
    s1 = pltpu.async_copy(gv_a, shared_ord.at[pos_a], sem)
    s2 = pltpu.async_copy(gv_b, shared_ord.at[pos_b], sem)
    m1.wait()
    s1.wait()
    s2.wait()
    plsc.subcore_barrier()

    pltpu.sync_copy(shared_ord.at[pl.ds(base, _CH)], ord_out.at[pl.ds(base, _CH)])


@jax.jit
def kernel(x):
    xf = x[0, :, 0]
    run = functools.partial(
        pl.kernel,
        out_type=(
            jax.ShapeDtypeStruct((_N,), _i32),
            jax.ShapeDtypeStruct((_N,), _i32),
            jax.ShapeDtypeStruct((_NB,), _i32),
        ),
        mesh=plsc.VectorSubcoreMesh(core_axis_name="c", subcore_axis_name="s"),
        scratch_types=[
            pltpu.VMEM((_CH,), jnp.float32),
            pltpu.VMEM((_CH,), _i32),
            pltpu.VMEM((_CH,), _i32),
            pltpu.VMEM((128,), _i32),
            pltpu.VMEM((128,), _i32),
            pltpu.VMEM((128,), _i32),
            pltpu.VMEM((128,), _i32),
            pltpu.VMEM((_NB,), _i32),
            pltpu.VMEM((_NT * _NB,), _i32),
            pltpu.VMEM_SHARED((_NT * _NB,), _i32),
            pltpu.VMEM_SHARED((_N,), _i32),
            pltpu.SemaphoreType.DMA,
        ],
    )(_body)
    memberships, order, counts = run(xf)
    return memberships, order, counts
